# baseline (speedup 1.0000x reference)
import functools

import jax
import jax.numpy as jnp
from jax import lax
from jax.experimental import pallas as pl
from jax.experimental.pallas import tpu as pltpu
from jax.experimental.pallas import tpu_sc as plsc

N_QUANT = 8
CODEBOOK_SIZE = 8192
CHUNK = 128
BATCH = 16
SEQ = 4096
TOKENS = BATCH * SEQ
NUM_WORKERS = 32
TOK_W = TOKENS // NUM_WORKERS
CT = 32
CROWS = CT * N_QUANT
BT = 128
NBLK = TOK_W // BT

_mesh = plsc.VectorSubcoreMesh(core_axis_name="c", subcore_axis_name="s")


@functools.partial(
    pl.kernel,
    mesh=_mesh,
    out_type=jax.ShapeDtypeStruct((BATCH, SEQ * N_QUANT, CHUNK), jnp.float32),
    scratch_types=(
        [pltpu.VMEM((BT, N_QUANT), jnp.int32) for _ in range(2)]
        + [pltpu.VMEM((2, 128), jnp.int32) for _ in range(2)]
        + [pltpu.VMEM((CROWS, CHUNK), jnp.float32) for _ in range(2)]
        + [pltpu.SemaphoreType.DMA for _ in range(6)]
    ),
    compiler_params=pltpu.CompilerParams(needs_layout_passes=False),
)
def _emb_lookup(tbl_hbm, x_hbm, out_hbm, xsA, xsB, ipA, ipB, rowsA, rowsB,
                gsemA, gsemB, xsemA, xsemB, osemA, osemB):
    wid = lax.axis_index("s") * 2 + lax.axis_index("c")
    b = wid // 2
    t_base = (wid % 2) * TOK_W

    lanes = lax.iota(jnp.int32, 16)

    def stage_x(blk, xs, xsem):
        pltpu.async_copy(x_hbm.at[b, pl.ds(t_base + blk * BT, BT)], xs, xsem)

    def drain_x(xs, xsem):
        pltpu.make_async_copy(x_hbm.at[0, pl.ds(0, BT)], xs, xsem).wait()

    def build_perm(k, xs, ip):
        for g in range(CROWS // 16):
            tg = 16 * g // 64
            i_vec = (16 * g) % 64 // 8 + lanes // 8
            row = k * CT + tg * 8 + lanes % 8
            vals = plsc.load_gather(xs, [row, i_vec])
            vals = vals + i_vec * CODEBOOK_SIZE
            ip[(16 * g) // 128, pl.ds((16 * g) % 128, 16)] = vals

    def fire_gathers(ip, rows, gsem):
        pltpu.async_copy(tbl_hbm.at[ip.at[0]], rows.at[pl.ds(0, 128)], gsem)
        pltpu.async_copy(tbl_hbm.at[ip.at[1]], rows.at[pl.ds(128, 128)], gsem)

    def drain_gather_fire_writebacks(c, rows, gsem, osem):
        pltpu.make_async_copy(tbl_hbm.at[pl.ds(0, CROWS)], rows, gsem).wait()
        r0 = t_base * N_QUANT + c * CROWS
        pltpu.async_copy(rows, out_hbm.at[b, pl.ds(r0, CROWS)], osem)

    def drain_writebacks(rows, osem):
        pltpu.make_async_copy(
            tbl_hbm.at[pl.ds(0, CROWS)], rows, osem
        ).wait()

    def process_block(blk, xs):
        for k in range(0, 4, 2):
            a = 4 * blk + k
            build_perm(k, xs, ipA)

            @pl.when(a > 0)
            def _():
                drain_writebacks(rowsA, osemA)
            fire_gathers(ipA, rowsA, gsemA)
            build_perm(k + 1, xs, ipB)

            @pl.when(a > 0)
            def _():
                drain_writebacks(rowsB, osemB)
            fire_gathers(ipB, rowsB, gsemB)
            drain_gather_fire_writebacks(a, rowsA, gsemA, osemA)
            drain_gather_fire_writebacks(a + 1, rowsB, gsemB, osemB)

    stage_x(0, xsA, xsemA)
    drain_x(xsA, xsemA)

    def body(g2, carry):
        blk = 2 * g2
        stage_x(blk + 1, xsB, xsemB)
        process_block(blk, xsA)
        drain_x(xsB, xsemB)

        @pl.when(g2 < NBLK // 2 - 1)
        def _():
            stage_x(blk + 2, xsA, xsemA)
        process_block(blk + 1, xsB)

        @pl.when(g2 < NBLK // 2 - 1)
        def _():
            drain_x(xsA, xsemA)
        return carry

    lax.fori_loop(0, NBLK // 2, body, 0)
    drain_writebacks(rowsA, osemA)
    drain_writebacks(rowsB, osemB)


def kernel(x, tables):
    tbl = tables.reshape(N_QUANT * CODEBOOK_SIZE, CHUNK)
    out3 = _emb_lookup(tbl, x.astype(jnp.int32))
    y5 = out3.reshape(BATCH, SEQ // 8, N_QUANT, 8, CHUNK)
    return y5.transpose(0, 1, 3, 2, 4).reshape(BATCH, SEQ, N_QUANT * CHUNK)


# speedup vs baseline: 1.0153x; 1.0153x over previous
import functools

import jax
import jax.numpy as jnp
from jax import lax
from jax.experimental import pallas as pl
from jax.experimental.pallas import tpu as pltpu
from jax.experimental.pallas import tpu_sc as plsc

N_QUANT = 8
CODEBOOK_SIZE = 8192
CHUNK = 128
BATCH = 16
SEQ = 4096
TOKENS = BATCH * SEQ
NUM_WORKERS = 32
TOK_W = TOKENS // NUM_WORKERS
CT = 32
CROWS = CT * N_QUANT
BT = 128
NBLK = TOK_W // BT

_mesh = plsc.VectorSubcoreMesh(core_axis_name="c", subcore_axis_name="s")


@functools.partial(
    pl.kernel,
    mesh=_mesh,
    out_type=jax.ShapeDtypeStruct((BATCH, SEQ * N_QUANT, CHUNK), jnp.float32),
    scratch_types=(
        [pltpu.VMEM((BT, N_QUANT), jnp.int32) for _ in range(2)]
        + [pltpu.VMEM((2, 128), jnp.int32) for _ in range(2)]
        + [pltpu.VMEM((CROWS, CHUNK), jnp.float32) for _ in range(2)]
        + [pltpu.SemaphoreType.DMA((2,)) for _ in range(2)]
        + [pltpu.SemaphoreType.DMA for _ in range(4)]
    ),
    compiler_params=pltpu.CompilerParams(needs_layout_passes=False),
)
def _emb_lookup(tbl_hbm, x_hbm, out_hbm, xsA, xsB, ipA, ipB, rowsA, rowsB,
                gsemA, gsemB, xsemA, xsemB, osemA, osemB):
    wid = lax.axis_index("s") * 2 + lax.axis_index("c")
    b = wid // 2
    t_base = (wid % 2) * TOK_W

    lanes = lax.iota(jnp.int32, 16)

    def stage_x(blk, xs, xsem):
        pltpu.async_copy(x_hbm.at[b, pl.ds(t_base + blk * BT, BT)], xs, xsem)

    def drain_x(xs, xsem):
        pltpu.make_async_copy(x_hbm.at[0, pl.ds(0, BT)], xs, xsem).wait()

    def build_perm(k, xs, ip):
        for g in range(CROWS // 16):
            tg = 16 * g // 64
            i_vec = (16 * g) % 64 // 8 + lanes // 8
            row = k * CT + tg * 8 + lanes % 8
            vals = plsc.load_gather(xs, [row, i_vec])
            vals = vals + i_vec * CODEBOOK_SIZE
            ip[(16 * g) // 128, pl.ds((16 * g) % 128, 16)] = vals

    def fire_gathers(ip, rows, gsem):
        pltpu.async_copy(tbl_hbm.at[ip.at[0]], rows.at[pl.ds(0, 128)],
                         gsem.at[0])
        pltpu.async_copy(tbl_hbm.at[ip.at[1]], rows.at[pl.ds(128, 128)],
                         gsem.at[1])

    def drain_gather_fire_writebacks(c, rows, gsem, osem):
        r0 = t_base * N_QUANT + c * CROWS
        for h in range(2):
            pltpu.make_async_copy(
                tbl_hbm.at[pl.ds(0, 128)], rows.at[pl.ds(h * 128, 128)],
                gsem.at[h],
            ).wait()
            pltpu.async_copy(
                rows.at[pl.ds(h * 128, 128)],
                out_hbm.at[b, pl.ds(r0 + h * 128, 128)],
                osem,
            )

    def drain_writebacks(rows, osem):
        pltpu.make_async_copy(
            tbl_hbm.at[pl.ds(0, CROWS)], rows, osem
        ).wait()

    def process_block(blk, xs):
        for k in range(0, 4, 2):
            a = 4 * blk + k
            build_perm(k, xs, ipA)

            @pl.when(a > 0)
            def _():
                drain_writebacks(rowsA, osemA)
            fire_gathers(ipA, rowsA, gsemA)
            build_perm(k + 1, xs, ipB)

            @pl.when(a > 0)
            def _():
                drain_writebacks(rowsB, osemB)
            fire_gathers(ipB, rowsB, gsemB)
            drain_gather_fire_writebacks(a, rowsA, gsemA, osemA)
            drain_gather_fire_writebacks(a + 1, rowsB, gsemB, osemB)

    stage_x(0, xsA, xsemA)
    drain_x(xsA, xsemA)

    def body(g2, carry):
        blk = 2 * g2
        stage_x(blk + 1, xsB, xsemB)
        process_block(blk, xsA)
        drain_x(xsB, xsemB)

        @pl.when(g2 < NBLK // 2 - 1)
        def _():
            stage_x(blk + 2, xsA, xsemA)
        process_block(blk + 1, xsB)

        @pl.when(g2 < NBLK // 2 - 1)
        def _():
            drain_x(xsA, xsemA)
        return carry

    lax.fori_loop(0, NBLK // 2, body, 0)
    drain_writebacks(rowsA, osemA)
    drain_writebacks(rowsB, osemB)


def kernel(x, tables):
    tbl = tables.reshape(N_QUANT * CODEBOOK_SIZE, CHUNK)
    out3 = _emb_lookup(tbl, x.astype(jnp.int32))
    y5 = out3.reshape(BATCH, SEQ // 8, N_QUANT, 8, CHUNK)
    return y5.transpose(0, 1, 3, 2, 4).reshape(BATCH, SEQ, N_QUANT * CHUNK)


# speedup vs baseline: 1.0468x; 1.0310x over previous
import functools

import jax
import jax.numpy as jnp
from jax import lax
from jax.experimental import pallas as pl
from jax.experimental.pallas import tpu as pltpu
from jax.experimental.pallas import tpu_sc as plsc

N_QUANT = 8
CODEBOOK_SIZE = 8192
CHUNK = 128
BATCH = 16
SEQ = 4096
TOKENS = BATCH * SEQ
NUM_WORKERS = 32
TOK_W = TOKENS // NUM_WORKERS
CT = 16
CROWS = CT * N_QUANT
NCH = TOK_W // CT
BT = 64
NBLK = TOK_W // BT
NSLOT = 4

_mesh = plsc.VectorSubcoreMesh(core_axis_name="c", subcore_axis_name="s")


@functools.partial(
    pl.kernel,
    mesh=_mesh,
    out_type=jax.ShapeDtypeStruct((BATCH, SEQ * N_QUANT, CHUNK), jnp.float32),
    scratch_types=(
        [pltpu.VMEM((BT, N_QUANT), jnp.int32) for _ in range(2)]
        + [pltpu.VMEM((CROWS,), jnp.int32) for _ in range(NSLOT)]
        + [pltpu.VMEM((CROWS, CHUNK), jnp.float32) for _ in range(NSLOT)]
        + [pltpu.SemaphoreType.DMA((NSLOT,)) for _ in range(2)]
        + [pltpu.SemaphoreType.DMA for _ in range(2)]
    ),
    compiler_params=pltpu.CompilerParams(needs_layout_passes=False),
)
def _emb_lookup(tbl_hbm, x_hbm, out_hbm, xsA, xsB,
                ip0, ip1, ip2, ip3, rv0, rv1, rv2, rv3,
                gsem, osem, xsemA, xsemB):
    ips = (ip0, ip1, ip2, ip3)
    rvs = (rv0, rv1, rv2, rv3)
    wid = lax.axis_index("s") * 2 + lax.axis_index("c")
    b = wid // 2
    t_base = (wid % 2) * TOK_W
    r_base = t_base * N_QUANT

    lanes = lax.iota(jnp.int32, 16)

    def stage_x(blk, xs, xsem):
        pltpu.async_copy(x_hbm.at[b, pl.ds(t_base + blk * BT, BT)], xs, xsem)

    def drain_x(xs, xsem):
        pltpu.make_async_copy(x_hbm.at[0, pl.ds(0, BT)], xs, xsem).wait()

    def build_perm(k, xs, ip):
        for g in range(CROWS // 16):
            tg = 16 * g // 64
            i_vec = (16 * g) % 64 // 8 + lanes // 8
            row = k * CT + tg * 8 + lanes % 8
            vals = plsc.load_gather(xs, [row, i_vec])
            ip[pl.ds(16 * g, 16)] = vals + i_vec * CODEBOOK_SIZE

    def fire_gather(s):
        pltpu.async_copy(tbl_hbm.at[ips[s]], rvs[s], gsem.at[s])

    def drain_gather(s):
        pltpu.make_async_copy(tbl_hbm.at[pl.ds(0, CROWS)], rvs[s],
                              gsem.at[s]).wait()

    def fire_writeback(c, s):
        pltpu.async_copy(rvs[s],
                         out_hbm.at[b, pl.ds(r_base + c * CROWS, CROWS)],
                         osem.at[s])

    def drain_writeback(s):
        pltpu.make_async_copy(tbl_hbm.at[pl.ds(0, CROWS)], rvs[s],
                              osem.at[s]).wait()

    stage_x(0, xsA, xsemA)
    drain_x(xsA, xsemA)
    stage_x(1, xsB, xsemB)
    build_perm(0, xsA, ips[0])
    fire_gather(0)
    build_perm(1, xsA, ips[1])
    fire_gather(1)

    def body(h, carry):
        c0 = 8 * h

        @pl.when(h > 0)
        def _():
            drain_writeback(2)
        build_perm(2, xsA, ips[2])
        fire_gather(2)
        drain_gather(0)
        fire_writeback(c0, 0)

        @pl.when(h > 0)
        def _():
            drain_writeback(3)
        build_perm(3, xsA, ips[3])
        fire_gather(3)

        @pl.when(h < NBLK // 2 - 1)
        def _():
            stage_x(2 * h + 2, xsA, xsemA)
        drain_gather(1)
        fire_writeback(c0 + 1, 1)

        drain_x(xsB, xsemB)
        drain_writeback(0)
        build_perm(0, xsB, ips[0])
        fire_gather(0)
        drain_gather(2)
        fire_writeback(c0 + 2, 2)

        drain_writeback(1)
        build_perm(1, xsB, ips[1])
        fire_gather(1)
        drain_gather(3)
        fire_writeback(c0 + 3, 3)

        drain_writeback(2)
        build_perm(2, xsB, ips[2])
        fire_gather(2)
        drain_gather(0)
        fire_writeback(c0 + 4, 0)

        drain_writeback(3)
        build_perm(3, xsB, ips[3])
        fire_gather(3)

        @pl.when(h < NBLK // 2 - 1)
        def _():
            stage_x(2 * h + 3, xsB, xsemB)
        drain_gather(1)
        fire_writeback(c0 + 5, 1)

        @pl.when(h < NBLK // 2 - 1)
        def _():
            drain_x(xsA, xsemA)
            drain_writeback(0)
            build_perm(0, xsA, ips[0])
            fire_gather(0)
        drain_gather(2)
        fire_writeback(c0 + 6, 2)

        @pl.when(h < NBLK // 2 - 1)
        def _():
            drain_writeback(1)
            build_perm(1, xsA, ips[1])
            fire_gather(1)
        drain_gather(3)
        fire_writeback(c0 + 7, 3)
        return carry

    lax.fori_loop(0, NBLK // 2, body, 0)
    for s in range(NSLOT):
        drain_writeback(s)


def kernel(x, tables):
    tbl = tables.reshape(N_QUANT * CODEBOOK_SIZE, CHUNK)
    out3 = _emb_lookup(tbl, x.astype(jnp.int32))
    y5 = out3.reshape(BATCH, SEQ // 8, N_QUANT, 8, CHUNK)
    return y5.transpose(0, 1, 3, 2, 4).reshape(BATCH, SEQ, N_QUANT * CHUNK)
